# Initial kernel scaffold; baseline (speedup 1.0000x reference)
import functools

import jax
import jax.numpy as jnp
import numpy as np
from jax import lax
from jax.experimental import pallas as pl
from jax.experimental.pallas import tpu as pltpu
from jax.experimental.pallas import tpu_sc as plsc

_V = 64
_E = 4
_B = _V * _E
_ROW = 3 * 224 * 224

_NC = 2
_NS = 16
_NW = _NC * _NS

_K = 24
_C = _ROW // _K
_XR = _V * _K
_OR = _B * _K
_RPW = _OR // _NW
_G = 8
_NIT = _RPW // _G

_TBLK = 16


def _make_index_tables():
    key = jax.random.key(42)
    base = jnp.arange(_V, dtype=jnp.int32)

    def shuf(k, x):
        return x.reshape(-1)[jax.random.permutation(k, x.size)]

    main = shuf(jax.random.fold_in(key, 0), base)
    thr = int(_V * 0.5)
    parts = [
        jnp.concatenate([shuf(jax.random.fold_in(key, i + 1), main[:thr]), main[thr:]])
        for i in range(_E)
    ]
    idx_flat = np.asarray(jnp.concatenate(parts), dtype=np.int32)
    gidx = (idx_flat[:, None] * _K + np.arange(_K, dtype=np.int32)[None, :]).reshape(-1)
    return gidx, idx_flat


@functools.cache
def _index_tables():
    return _make_index_tables()


def _sc_body(x_hbm, gidx_hbm, tgt_hbm, tidx_hbm, out_hbm, tout_hbm,
             idx_v, b0, b1, t_tab, t_idx, t_out, si0, si1, so0, so1):
    wid = lax.axis_index("s") * _NC + lax.axis_index("c")
    rbase = wid * _RPW

    pltpu.sync_copy(gidx_hbm.at[pl.ds(rbase, _RPW)], idx_v)

    bufs = (b0, b1)
    sin = (si0, si1)
    sout = (so0, so1)
    in_h = [None, None]
    out_h = [None, None]

    in_h[0] = pltpu.async_copy(x_hbm.at[idx_v.at[pl.ds(0, _G)]], b0, si0)
    for j in range(_NIT):
        s = j & 1
        nj = j + 1
        if nj < _NIT:
            ns = nj & 1
            if out_h[ns] is not None:
                out_h[ns].wait()
            in_h[ns] = pltpu.async_copy(
                x_hbm.at[idx_v.at[pl.ds(nj * _G, _G)]], bufs[ns], sin[ns])
        in_h[s].wait()
        out_h[s] = pltpu.async_copy(
            bufs[s], out_hbm.at[pl.ds(rbase + j * _G, _G)], sout[s])
    out_h[0].wait()
    out_h[1].wait()

    @pl.when(wid == 0)
    def _targets():
        pltpu.sync_copy(tgt_hbm, t_tab)
        pltpu.sync_copy(tidx_hbm, t_idx)
        for j in range(_B // _TBLK):
            iv = t_idx[pl.ds(j * _TBLK, _TBLK)]
            t_out[pl.ds(j * _TBLK, _TBLK)] = plsc.load_gather(t_tab, [iv])
        pltpu.sync_copy(t_out, tout_hbm)


@functools.cache
def _sc_call():
    mesh = plsc.VectorSubcoreMesh(
        core_axis_name="c", subcore_axis_name="s",
        num_cores=_NC, num_subcores=_NS)
    return pl.kernel(
        _sc_body,
        out_type=(
            jax.ShapeDtypeStruct((_OR, _C), jnp.float32),
            jax.ShapeDtypeStruct((_B,), jnp.int32),
        ),
        mesh=mesh,
        scratch_types=[
            pltpu.VMEM((_RPW,), jnp.int32),
            pltpu.VMEM((_G, _C), jnp.float32),
            pltpu.VMEM((_G, _C), jnp.float32),
            pltpu.VMEM((_V,), jnp.int32),
            pltpu.VMEM((_B,), jnp.int32),
            pltpu.VMEM((_B,), jnp.int32),
            pltpu.SemaphoreType.DMA,
            pltpu.SemaphoreType.DMA,
            pltpu.SemaphoreType.DMA,
            pltpu.SemaphoreType.DMA,
        ],
    )


def kernel(inputs, targets):
    gidx, tidx = _index_tables()
    x = inputs.reshape(_XR, _C)
    out2, tout = _sc_call()(x, jnp.asarray(gidx), targets, jnp.asarray(tidx))
    return out2.reshape(_B, 3, 224, 224), tout


# speedup vs baseline: 1.0769x; 1.0769x over previous
import functools

import jax
import jax.numpy as jnp
import numpy as np
from jax import lax
from jax.experimental import pallas as pl
from jax.experimental.pallas import tpu as pltpu
from jax.experimental.pallas import tpu_sc as plsc

_V = 64
_E = 4
_B = _V * _E
_ROW = 3 * 224 * 224

_NC = 2
_NS = 16
_NW = _NC * _NS

_K = 24
_C = _ROW // _K
_XR = _V * _K
_OR = _B * _K
_RPW = _OR // _NW
_G = 8
_NIT = _RPW // _G

_TBLK = 128


_IDX_FLAT = np.array([
    [42, 45, 52, 14, 38, 17, 1, 47, 19, 50, 5, 9, 39, 20, 15, 31, 44, 3, 0,
     49, 51, 61, 28, 33, 58, 32, 11, 27, 40, 54, 46, 2, 36, 35, 62, 63, 21,
     59, 30, 43, 22, 18, 24, 26, 53, 12, 16, 6, 7, 57, 55, 48, 13, 37, 60,
     10, 29, 34, 25, 56, 4, 41, 23, 8],
    [39, 50, 54, 44, 3, 51, 52, 17, 27, 1, 14, 38, 42, 33, 9, 58, 46, 32, 40,
     49, 47, 19, 2, 31, 15, 11, 20, 5, 61, 0, 45, 28, 36, 35, 62, 63, 21, 59,
     30, 43, 22, 18, 24, 26, 53, 12, 16, 6, 7, 57, 55, 48, 13, 37, 60, 10,
     29, 34, 25, 56, 4, 41, 23, 8],
    [45, 1, 5, 3, 61, 49, 32, 38, 42, 2, 39, 52, 47, 44, 0, 19, 54, 50, 46,
     9, 14, 31, 51, 58, 15, 17, 11, 33, 27, 28, 40, 20, 36, 35, 62, 63, 21,
     59, 30, 43, 22, 18, 24, 26, 53, 12, 16, 6, 7, 57, 55, 48, 13, 37, 60,
     10, 29, 34, 25, 56, 4, 41, 23, 8],
    [58, 45, 15, 33, 3, 38, 19, 31, 27, 28, 49, 32, 42, 54, 50, 11, 51, 52,
     40, 5, 1, 9, 44, 61, 14, 0, 2, 17, 47, 20, 39, 46, 36, 35, 62, 63, 21,
     59, 30, 43, 22, 18, 24, 26, 53, 12, 16, 6, 7, 57, 55, 48, 13, 37, 60,
     10, 29, 34, 25, 56, 4, 41, 23, 8],
], dtype=np.int32).reshape(-1)


@functools.cache
def _index_tables():
    idx_flat = _IDX_FLAT
    gidx = (idx_flat[:, None] * _K + np.arange(_K, dtype=np.int32)[None, :]).reshape(-1)
    return gidx, idx_flat


def _sc_body(x_hbm, gidx_hbm, tgt_hbm, tidx_hbm, out_hbm, tout_hbm,
             idx_v, b0, b1, t_idx, t_buf, si0, si1, so0, so1, tsem):
    wid = lax.axis_index("s") * _NC + lax.axis_index("c")
    rbase = wid * _RPW

    pltpu.sync_copy(gidx_hbm.at[pl.ds(rbase, _RPW)], idx_v)

    bufs = (b0, b1)
    sin = (si0, si1)
    sout = (so0, so1)
    in_h = [None, None]
    out_h = [None, None]

    in_h[0] = pltpu.async_copy(x_hbm.at[idx_v.at[pl.ds(0, _G)]], b0, si0)
    for j in range(_NIT):
        s = j & 1
        nj = j + 1
        if nj < _NIT:
            ns = nj & 1
            if out_h[ns] is not None:
                out_h[ns].wait()
            in_h[ns] = pltpu.async_copy(
                x_hbm.at[idx_v.at[pl.ds(nj * _G, _G)]], bufs[ns], sin[ns])
        in_h[s].wait()
        out_h[s] = pltpu.async_copy(
            bufs[s], out_hbm.at[pl.ds(rbase + j * _G, _G)], sout[s])
    out_h[0].wait()
    out_h[1].wait()

    @pl.when(wid == 0)
    def _targets():
        pltpu.sync_copy(tidx_hbm, t_idx)
        for h in range(2):
            pltpu.async_copy(
                tgt_hbm.at[t_idx.at[pl.ds(h * 128, 128)]], t_buf, tsem).wait()
            pltpu.sync_copy(t_buf, tout_hbm.at[pl.ds(h * 128, 128)])


@functools.cache
def _sc_call():
    mesh = plsc.VectorSubcoreMesh(
        core_axis_name="c", subcore_axis_name="s",
        num_cores=_NC, num_subcores=_NS)
    return pl.kernel(
        _sc_body,
        out_type=(
            jax.ShapeDtypeStruct((_OR, _C), jnp.float32),
            jax.ShapeDtypeStruct((_B, _TBLK), jnp.int32),
        ),
        mesh=mesh,
        scratch_types=[
            pltpu.VMEM((_RPW,), jnp.int32),
            pltpu.VMEM((_G, _C), jnp.float32),
            pltpu.VMEM((_G, _C), jnp.float32),
            pltpu.VMEM((_B,), jnp.int32),
            pltpu.VMEM((128, _TBLK), jnp.int32),
            pltpu.SemaphoreType.DMA,
            pltpu.SemaphoreType.DMA,
            pltpu.SemaphoreType.DMA,
            pltpu.SemaphoreType.DMA,
            pltpu.SemaphoreType.DMA,
        ],
    )


def kernel(inputs, targets):
    gidx, tidx = _index_tables()
    x = inputs.reshape(_XR, _C)
    tgt2 = jnp.broadcast_to(targets[:, None], (_V, _TBLK))
    out2, tout2 = _sc_call()(x, jnp.asarray(gidx), tgt2, jnp.asarray(tidx))
    return out2.reshape(_B, 3, 224, 224), tout2[:, 0]


# speedup vs baseline: 1.1791x; 1.0949x over previous
import functools

import jax
import jax.numpy as jnp
import numpy as np
from jax import lax
from jax.experimental import pallas as pl
from jax.experimental.pallas import tpu as pltpu
from jax.experimental.pallas import tpu_sc as plsc

_V = 64
_E = 4
_B = _V * _E
_ROW = 3 * 224 * 224

_NC = 2
_NS = 16
_NW = _NC * _NS

_K = 24
_C = _ROW // _K
_XR = _V * _K
_OR = _B * _K
_RPW = _XR // _NW
_G = 8
_NIT = _RPW // _G

_TBLK = 128


_IDX_FLAT = np.array([
    [42, 45, 52, 14, 38, 17, 1, 47, 19, 50, 5, 9, 39, 20, 15, 31, 44, 3, 0,
     49, 51, 61, 28, 33, 58, 32, 11, 27, 40, 54, 46, 2, 36, 35, 62, 63, 21,
     59, 30, 43, 22, 18, 24, 26, 53, 12, 16, 6, 7, 57, 55, 48, 13, 37, 60,
     10, 29, 34, 25, 56, 4, 41, 23, 8],
    [39, 50, 54, 44, 3, 51, 52, 17, 27, 1, 14, 38, 42, 33, 9, 58, 46, 32, 40,
     49, 47, 19, 2, 31, 15, 11, 20, 5, 61, 0, 45, 28, 36, 35, 62, 63, 21, 59,
     30, 43, 22, 18, 24, 26, 53, 12, 16, 6, 7, 57, 55, 48, 13, 37, 60, 10,
     29, 34, 25, 56, 4, 41, 23, 8],
    [45, 1, 5, 3, 61, 49, 32, 38, 42, 2, 39, 52, 47, 44, 0, 19, 54, 50, 46,
     9, 14, 31, 51, 58, 15, 17, 11, 33, 27, 28, 40, 20, 36, 35, 62, 63, 21,
     59, 30, 43, 22, 18, 24, 26, 53, 12, 16, 6, 7, 57, 55, 48, 13, 37, 60,
     10, 29, 34, 25, 56, 4, 41, 23, 8],
    [58, 45, 15, 33, 3, 38, 19, 31, 27, 28, 49, 32, 42, 54, 50, 11, 51, 52,
     40, 5, 1, 9, 44, 61, 14, 0, 2, 17, 47, 20, 39, 46, 36, 35, 62, 63, 21,
     59, 30, 43, 22, 18, 24, 26, 53, 12, 16, 6, 7, 57, 55, 48, 13, 37, 60,
     10, 29, 34, 25, 56, 4, 41, 23, 8],
], dtype=np.int32)


@functools.cache
def _index_tables():
    idx_flat = _IDX_FLAT.reshape(-1)
    inv = np.empty((_E, _V), np.int32)
    for e in range(_E):
        inv[e, _IDX_FLAT[e]] = np.arange(_V, dtype=np.int32)
    r = np.arange(_XR, dtype=np.int32)
    v, k = r // _K, r % _K
    dst = (inv[:, v] + np.arange(_E, dtype=np.int32)[:, None] * _V) * _K + k[None, :]
    stab = np.empty((_NW, _E * _NIT, _G), np.int32)
    for w in range(_NW):
        rows = w * _RPW + np.arange(_RPW)
        for e in range(_E):
            for j in range(_NIT):
                stab[w, e * _NIT + j] = dst[e, rows[j * _G:(j + 1) * _G]]
    return stab, idx_flat


def _sc_body(x_hbm, stab_hbm, tgt_hbm, tidx_hbm, out_hbm, tout_hbm,
             sidx_v, b0, b1, t_idx, t_buf, si0, si1, so0, so1, tsem):
    wid = lax.axis_index("s") * _NC + lax.axis_index("c")
    ibase = wid * _RPW

    pltpu.sync_copy(stab_hbm.at[wid], sidx_v)

    bufs = (b0, b1)
    sin = (si0, si1)
    sout = (so0, so1)
    in_h = [None, None]
    out_h = [None, None]

    def read(j, buf, sem):
        return pltpu.async_copy(x_hbm.at[pl.ds(ibase + j * _G, _G)], buf, sem)

    in_h[0] = read(0, b0, si0)
    for j in range(_NIT):
        s = j & 1
        nj = j + 1
        if nj < _NIT:
            ns = nj & 1
            if out_h[ns] is not None:
                for h in out_h[ns]:
                    h.wait()
            in_h[ns] = read(nj, bufs[ns], sin[ns])
        in_h[s].wait()
        out_h[s] = [
            pltpu.async_copy(bufs[s], out_hbm.at[sidx_v.at[e * _NIT + j]], sout[s])
            for e in range(_E)
        ]
    for s in (0, 1):
        for h in out_h[s]:
            h.wait()

    @pl.when(wid == 0)
    def _targets():
        pltpu.sync_copy(tidx_hbm, t_idx)
        for h in range(2):
            pltpu.async_copy(
                tgt_hbm.at[t_idx.at[pl.ds(h * 128, 128)]], t_buf, tsem).wait()
            pltpu.sync_copy(t_buf, tout_hbm.at[pl.ds(h * 128, 128)])


@functools.cache
def _sc_call():
    mesh = plsc.VectorSubcoreMesh(
        core_axis_name="c", subcore_axis_name="s",
        num_cores=_NC, num_subcores=_NS)
    return pl.kernel(
        _sc_body,
        out_type=(
            jax.ShapeDtypeStruct((_OR, _C), jnp.float32),
            jax.ShapeDtypeStruct((_B, _TBLK), jnp.int32),
        ),
        mesh=mesh,
        scratch_types=[
            pltpu.VMEM((_E * _NIT, _G), jnp.int32),
            pltpu.VMEM((_G, _C), jnp.float32),
            pltpu.VMEM((_G, _C), jnp.float32),
            pltpu.VMEM((_B,), jnp.int32),
            pltpu.VMEM((128, _TBLK), jnp.int32),
            pltpu.SemaphoreType.DMA,
            pltpu.SemaphoreType.DMA,
            pltpu.SemaphoreType.DMA,
            pltpu.SemaphoreType.DMA,
            pltpu.SemaphoreType.DMA,
        ],
    )


def kernel(inputs, targets):
    stab, tidx = _index_tables()
    x = inputs.reshape(_XR, _C)
    tgt2 = jnp.broadcast_to(targets[:, None], (_V, _TBLK))
    out2, tout2 = _sc_call()(x, jnp.asarray(stab), tgt2, jnp.asarray(tidx))
    return out2.reshape(_B, 3, 224, 224), tout2[:, 0]


# speedup vs baseline: 1.2561x; 1.0653x over previous
import functools

import jax
import jax.numpy as jnp
import numpy as np
from jax import lax
from jax.experimental import pallas as pl
from jax.experimental.pallas import tpu as pltpu
from jax.experimental.pallas import tpu_sc as plsc

_V = 64
_E = 4
_B = _V * _E
_ROW = 3 * 224 * 224

_NC = 2
_NS = 16
_NW = _NC * _NS

_K = 3
_C = _ROW // _K
_XR = _V * _K
_OR = _B * _K
_RPW = _XR // _NW
_G = 1
_NIT = _RPW // _G

_TBLK = 128


_IDX_FLAT = np.array([
    [42, 45, 52, 14, 38, 17, 1, 47, 19, 50, 5, 9, 39, 20, 15, 31, 44, 3, 0,
     49, 51, 61, 28, 33, 58, 32, 11, 27, 40, 54, 46, 2, 36, 35, 62, 63, 21,
     59, 30, 43, 22, 18, 24, 26, 53, 12, 16, 6, 7, 57, 55, 48, 13, 37, 60,
     10, 29, 34, 25, 56, 4, 41, 23, 8],
    [39, 50, 54, 44, 3, 51, 52, 17, 27, 1, 14, 38, 42, 33, 9, 58, 46, 32, 40,
     49, 47, 19, 2, 31, 15, 11, 20, 5, 61, 0, 45, 28, 36, 35, 62, 63, 21, 59,
     30, 43, 22, 18, 24, 26, 53, 12, 16, 6, 7, 57, 55, 48, 13, 37, 60, 10,
     29, 34, 25, 56, 4, 41, 23, 8],
    [45, 1, 5, 3, 61, 49, 32, 38, 42, 2, 39, 52, 47, 44, 0, 19, 54, 50, 46,
     9, 14, 31, 51, 58, 15, 17, 11, 33, 27, 28, 40, 20, 36, 35, 62, 63, 21,
     59, 30, 43, 22, 18, 24, 26, 53, 12, 16, 6, 7, 57, 55, 48, 13, 37, 60,
     10, 29, 34, 25, 56, 4, 41, 23, 8],
    [58, 45, 15, 33, 3, 38, 19, 31, 27, 28, 49, 32, 42, 54, 50, 11, 51, 52,
     40, 5, 1, 9, 44, 61, 14, 0, 2, 17, 47, 20, 39, 46, 36, 35, 62, 63, 21,
     59, 30, 43, 22, 18, 24, 26, 53, 12, 16, 6, 7, 57, 55, 48, 13, 37, 60,
     10, 29, 34, 25, 56, 4, 41, 23, 8],
], dtype=np.int32)


@functools.cache
def _index_tables():
    idx_flat = _IDX_FLAT.reshape(-1)
    inv = np.empty((_E, _V), np.int32)
    for e in range(_E):
        inv[e, _IDX_FLAT[e]] = np.arange(_V, dtype=np.int32)
    r = np.arange(_XR, dtype=np.int32)
    v, k = r // _K, r % _K
    dst = (inv[:, v] + np.arange(_E, dtype=np.int32)[:, None] * _V) * _K + k[None, :]
    stab = np.empty((_NW, _E * _NIT, _G), np.int32)
    for w in range(_NW):
        rows = w * _RPW + np.arange(_RPW)
        for e in range(_E):
            for j in range(_NIT):
                stab[w, e * _NIT + j] = dst[e, rows[j * _G:(j + 1) * _G]]
    return stab, idx_flat


def _sc_body(x_hbm, stab_hbm, tgt_hbm, tidx_hbm, out_hbm, tout_hbm,
             sidx_v, b0, b1, t_idx, t_buf, si0, si1, so0, so1, tsem):
    wid = lax.axis_index("s") * _NC + lax.axis_index("c")
    ibase = wid * _RPW

    pltpu.sync_copy(stab_hbm.at[wid], sidx_v)

    bufs = (b0, b1)
    sin = (si0, si1)
    sout = (so0, so1)
    in_h = [None, None]
    out_h = [None, None]

    def read(j, buf, sem):
        return pltpu.async_copy(x_hbm.at[pl.ds(ibase + j * _G, _G)], buf, sem)

    in_h[0] = read(0, b0, si0)
    for j in range(_NIT):
        s = j & 1
        nj = j + 1
        if nj < _NIT:
            ns = nj & 1
            if out_h[ns] is not None:
                for h in out_h[ns]:
                    h.wait()
            in_h[ns] = read(nj, bufs[ns], sin[ns])
        in_h[s].wait()
        out_h[s] = [
            pltpu.async_copy(bufs[s], out_hbm.at[sidx_v.at[e * _NIT + j]], sout[s])
            for e in range(_E)
        ]
    for s in (0, 1):
        for h in out_h[s]:
            h.wait()

    @pl.when(wid == 0)
    def _targets():
        pltpu.sync_copy(tidx_hbm, t_idx)
        for h in range(2):
            pltpu.async_copy(
                tgt_hbm.at[t_idx.at[pl.ds(h * 128, 128)]], t_buf, tsem).wait()
            pltpu.sync_copy(t_buf, tout_hbm.at[pl.ds(h * 128, 128)])


@functools.cache
def _sc_call():
    mesh = plsc.VectorSubcoreMesh(
        core_axis_name="c", subcore_axis_name="s",
        num_cores=_NC, num_subcores=_NS)
    return pl.kernel(
        _sc_body,
        out_type=(
            jax.ShapeDtypeStruct((_OR, _C), jnp.float32),
            jax.ShapeDtypeStruct((_B, _TBLK), jnp.int32),
        ),
        mesh=mesh,
        scratch_types=[
            pltpu.VMEM((_E * _NIT, _G), jnp.int32),
            pltpu.VMEM((_G, _C), jnp.float32),
            pltpu.VMEM((_G, _C), jnp.float32),
            pltpu.VMEM((_B,), jnp.int32),
            pltpu.VMEM((128, _TBLK), jnp.int32),
            pltpu.SemaphoreType.DMA,
            pltpu.SemaphoreType.DMA,
            pltpu.SemaphoreType.DMA,
            pltpu.SemaphoreType.DMA,
            pltpu.SemaphoreType.DMA,
        ],
    )


def kernel(inputs, targets):
    stab, tidx = _index_tables()
    x = inputs.reshape(_XR, _C)
    tgt2 = jnp.broadcast_to(targets[:, None], (_V, _TBLK))
    out2, tout2 = _sc_call()(x, jnp.asarray(stab), tgt2, jnp.asarray(tidx))
    return out2.reshape(_B, 3, 224, 224), tout2[:, 0]


# speedup vs baseline: 1.6362x; 1.3026x over previous
import functools

import jax
import jax.numpy as jnp
import numpy as np
from jax import lax
from jax.experimental import pallas as pl
from jax.experimental.pallas import tpu as pltpu
from jax.experimental.pallas import tpu_sc as plsc

_V = 64
_E = 4
_B = _V * _E
_ROW = 3 * 224 * 224

_NC = 2
_VPC = _V // _NC
_GR = 4
_NG = _VPC // _GR

_TBLK = 128


_IDX_FLAT = np.array([
    [42, 45, 52, 14, 38, 17, 1, 47, 19, 50, 5, 9, 39, 20, 15, 31, 44, 3, 0,
     49, 51, 61, 28, 33, 58, 32, 11, 27, 40, 54, 46, 2, 36, 35, 62, 63, 21,
     59, 30, 43, 22, 18, 24, 26, 53, 12, 16, 6, 7, 57, 55, 48, 13, 37, 60,
     10, 29, 34, 25, 56, 4, 41, 23, 8],
    [39, 50, 54, 44, 3, 51, 52, 17, 27, 1, 14, 38, 42, 33, 9, 58, 46, 32, 40,
     49, 47, 19, 2, 31, 15, 11, 20, 5, 61, 0, 45, 28, 36, 35, 62, 63, 21, 59,
     30, 43, 22, 18, 24, 26, 53, 12, 16, 6, 7, 57, 55, 48, 13, 37, 60, 10,
     29, 34, 25, 56, 4, 41, 23, 8],
    [45, 1, 5, 3, 61, 49, 32, 38, 42, 2, 39, 52, 47, 44, 0, 19, 54, 50, 46,
     9, 14, 31, 51, 58, 15, 17, 11, 33, 27, 28, 40, 20, 36, 35, 62, 63, 21,
     59, 30, 43, 22, 18, 24, 26, 53, 12, 16, 6, 7, 57, 55, 48, 13, 37, 60,
     10, 29, 34, 25, 56, 4, 41, 23, 8],
    [58, 45, 15, 33, 3, 38, 19, 31, 27, 28, 49, 32, 42, 54, 50, 11, 51, 52,
     40, 5, 1, 9, 44, 61, 14, 0, 2, 17, 47, 20, 39, 46, 36, 35, 62, 63, 21,
     59, 30, 43, 22, 18, 24, 26, 53, 12, 16, 6, 7, 57, 55, 48, 13, 37, 60,
     10, 29, 34, 25, 56, 4, 41, 23, 8],
], dtype=np.int32)


@functools.cache
def _dest_rows():
    inv = np.empty((_E, _V), np.int64)
    for e in range(_E):
        inv[e, _IDX_FLAT[e]] = np.arange(_V)
    return [[int(e * _V + inv[e, v]) for e in range(_E)] for v in range(_V)]


def _core_pipeline(cid, x_hbm, tgt_hbm, out_hbm, tout_hbm,
                   b0, b1, tgt_s, si0, si1, so0, so1, tsem):
    dests = _dest_rows()
    vbase = cid * _VPC
    bufs = (b0, b1)
    sin = (si0, si1)
    sout = (so0, so1)
    in_h = [None, None]
    out_h = [None, None]

    def read(g, buf, sem):
        return pltpu.async_copy(x_hbm.at[pl.ds(vbase + g * _GR, _GR)], buf, sem)

    pltpu.sync_copy(tgt_hbm, tgt_s)
    t_h = []
    for v in range(vbase, vbase + _VPC):
        for d in dests[v]:
            t_h.append(pltpu.async_copy(tgt_s.at[v], tout_hbm.at[d], tsem))

    in_h[0] = read(0, b0, si0)
    for g in range(_NG):
        s = g & 1
        ng = g + 1
        if ng < _NG:
            ns = ng & 1
            if out_h[ns] is not None:
                for h in out_h[ns]:
                    h.wait()
            in_h[ns] = read(ng, bufs[ns], sin[ns])
        in_h[s].wait()
        hs = []
        for r in range(_GR):
            v = vbase + g * _GR + r
            for d in dests[v]:
                hs.append(pltpu.async_copy(bufs[s].at[r], out_hbm.at[d], sout[s]))
        out_h[s] = hs
    for s in (0, 1):
        for h in out_h[s]:
            h.wait()
    for h in t_h:
        h.wait()


def _sc_body(x_hbm, tgt_hbm, out_hbm, tout_hbm,
             b0, b1, tgt_s, si0, si1, so0, so1, tsem):
    cid = lax.axis_index("c")
    for c in range(_NC):
        @pl.when(cid == c)
        def _(c=c):
            _core_pipeline(c, x_hbm, tgt_hbm, out_hbm, tout_hbm,
                           b0, b1, tgt_s, si0, si1, so0, so1, tsem)


@functools.cache
def _sc_call():
    mesh = plsc.ScalarSubcoreMesh(axis_name="c", num_cores=_NC)
    return pl.kernel(
        _sc_body,
        out_type=(
            jax.ShapeDtypeStruct((_B, _ROW), jnp.float32),
            jax.ShapeDtypeStruct((_B, _TBLK), jnp.int32),
        ),
        mesh=mesh,
        scratch_types=[
            pltpu.VMEM_SHARED((_GR, _ROW), jnp.float32),
            pltpu.VMEM_SHARED((_GR, _ROW), jnp.float32),
            pltpu.VMEM_SHARED((_V, _TBLK), jnp.int32),
            pltpu.SemaphoreType.DMA,
            pltpu.SemaphoreType.DMA,
            pltpu.SemaphoreType.DMA,
            pltpu.SemaphoreType.DMA,
            pltpu.SemaphoreType.DMA,
        ],
    )


def kernel(inputs, targets):
    x = inputs.reshape(_V, _ROW)
    tgt2 = jnp.broadcast_to(targets[:, None], (_V, _TBLK))
    out2, tout2 = _sc_call()(x, tgt2)
    return out2.reshape(_B, 3, 224, 224), tout2[:, 0]
